# baseline (speedup 1.0000x reference)
import functools

import jax
import jax.numpy as jnp
from jax import lax
from jax.experimental import pallas as pl
from jax.experimental.pallas import tpu as pltpu
from jax.experimental.pallas import tpu_sc as plsc

B, S, D = 4, 4096, 2048

_INFO = plsc.get_sparse_core_info()
NC, NS, L = _INFO.num_cores, _INFO.num_subcores, _INFO.num_lanes
NW = NC * NS
SPW = S // NW
R = 8
NCHUNK = SPW // R

_mesh = plsc.VectorSubcoreMesh(core_axis_name="c", subcore_axis_name="s")


def _body(x_hbm, tbl_hbm, out_hbm, tbuf, xbuf):
    wid = lax.axis_index("s") * NC + lax.axis_index("c")
    s_base = wid * SPW

    def chunk(c, carry):
        s0 = s_base + c * R
        pltpu.sync_copy(tbl_hbm.at[pl.ds(s0, R)], tbuf)

        def bloop(b, carry2):
            row0 = b * S + s0
            pltpu.sync_copy(x_hbm.at[pl.ds(row0, R)], xbuf)

            def kloop(k, carry3):
                sl = pl.ds(k * L, L)
                for r in range(R):
                    xbuf[r, sl] = xbuf[r, sl] + tbuf[r, sl]
                return carry3

            lax.fori_loop(0, D // L, kloop, 0)
            pltpu.sync_copy(xbuf, out_hbm.at[pl.ds(row0, R)])
            return carry2

        lax.fori_loop(0, B, bloop, 0)
        return carry

    lax.fori_loop(0, NCHUNK, chunk, 0)


@jax.jit
def kernel(encoded_tokens, position_table):
    x = encoded_tokens.reshape(B * S, D)
    run = pl.kernel(
        _body,
        out_type=jax.ShapeDtypeStruct((B * S, D), jnp.float32),
        mesh=_mesh,
        scratch_types=[
            pltpu.VMEM((R, D), jnp.float32),
            pltpu.VMEM((R, D), jnp.float32),
        ],
    )
    out = run(x, position_table)
    return out.reshape(B, S, D)
